# baseline (speedup 1.0000x reference)
import functools

import jax
import jax.numpy as jnp
from jax import lax
from jax.experimental import pallas as pl
from jax.experimental.pallas import tpu as pltpu
from jax.experimental.pallas import tpu_sc as plsc

B, S, H, E = 4, 8192, 2048, 64
NC, NS = 2, 16
NW = NC * NS
WPB = NW // B
HC = H // WPB
NV = HC // 16
R = 128
NCHUNK = S // R


def _pool_body(hs, out, buf, accv, sem0, sem1):
    wid = lax.axis_index("s") * NC + lax.axis_index("c")
    b = wid // WPB
    h0 = (wid % WPB) * HC

    def copy_in(g, slot, sem):
        return pltpu.make_async_copy(
            hs.at[b, pl.ds(g * R, R), pl.ds(h0, HC)], buf.at[slot], sem)

    def acc_rows(slot, acc):
        def row_step(r, a):
            return tuple(a[h] + buf[slot, r, pl.ds(h * 16, 16)]
                         for h in range(NV))
        return lax.fori_loop(0, R, row_step, acc)

    copy_in(0, 0, sem0).start()
    acc0 = tuple(jnp.zeros((16,), jnp.float32) for _ in range(NV))

    def chunk_pair(i, acc):
        g = i * 2
        copy_in(g + 1, 1, sem1).start()
        copy_in(g, 0, sem0).wait()
        acc = acc_rows(0, acc)

        @pl.when(g + 2 < NCHUNK)
        def _():
            copy_in(g + 2, 0, sem0).start()

        copy_in(g + 1, 1, sem1).wait()
        acc = acc_rows(1, acc)
        return acc

    acc = lax.fori_loop(0, NCHUNK // 2, chunk_pair, acc0)
    for h in range(NV):
        accv[pl.ds(h * 16, 16)] = acc[h] * (1.0 / S)
    pltpu.sync_copy(accv, out.at[b, pl.ds(h0, HC)])


_pool = functools.partial(
    pl.kernel,
    mesh=plsc.VectorSubcoreMesh(core_axis_name="c", subcore_axis_name="s"),
    out_type=jax.ShapeDtypeStruct((B, H), jnp.float32),
    scratch_types=[
        pltpu.VMEM((2, R, HC), jnp.float32),
        pltpu.VMEM((HC,), jnp.float32),
        pltpu.SemaphoreType.DMA,
        pltpu.SemaphoreType.DMA,
    ],
)(_pool_body)


def _tail_body(pooled_ref, w_ref, bias_ref, temp_ref, noise_ref, gum_ref,
               sel_ref, probs_ref):
    pooled = pooled_ref[...]
    w = w_ref[...]
    logits = lax.dot_general(pooled, w, (((1,), (0,)), ((), ())),
                             preferred_element_type=jnp.float32)
    logits = logits + bias_ref[...] + noise_ref[...]
    t = temp_ref[0, 0]
    x = logits / t
    x = x - jnp.max(x, axis=-1, keepdims=True)
    ex = jnp.exp(x)
    probs = ex / jnp.sum(ex, axis=-1, keepdims=True)
    probs_ref[...] = probs
    y = jnp.log(probs + 1e-20) + gum_ref[...]
    idx = lax.broadcasted_iota(jnp.int32, y.shape, 1)
    m1 = jnp.max(y, axis=-1, keepdims=True)
    i1 = jnp.min(jnp.where(y == m1, idx, E), axis=-1, keepdims=True)
    y2 = jnp.where(idx == i1, -jnp.inf, y)
    m2 = jnp.max(y2, axis=-1, keepdims=True)
    i2 = jnp.min(jnp.where(y2 == m2, idx, E), axis=-1, keepdims=True)
    sel_ref[...] = jnp.concatenate([i1, i2], axis=-1)


_tail = pl.pallas_call(
    _tail_body,
    out_shape=(jax.ShapeDtypeStruct((B, 2), jnp.int32),
               jax.ShapeDtypeStruct((B, E), jnp.float32)),
    in_specs=[
        pl.BlockSpec(memory_space=pltpu.VMEM),
        pl.BlockSpec(memory_space=pltpu.VMEM),
        pl.BlockSpec(memory_space=pltpu.VMEM),
        pl.BlockSpec(memory_space=pltpu.SMEM),
        pl.BlockSpec(memory_space=pltpu.VMEM),
        pl.BlockSpec(memory_space=pltpu.VMEM),
    ],
)


def kernel(hidden_states, W, b, temperature, diversity_bonus):
    noise_key = jax.random.fold_in(jax.random.key(0), 1)
    noise = jax.random.normal(noise_key, (B, E), jnp.float32) * diversity_bonus
    g_key = jax.random.fold_in(jax.random.key(0), 2)
    gumbel = jax.random.gumbel(g_key, (B, E), jnp.float32)
    pooled = _pool(hidden_states)
    sel, probs = _tail(pooled, W, b.reshape(1, E),
                       temperature.reshape(1, 1), noise, gumbel)
    return sel, probs


# speedup vs baseline: 1.2384x; 1.2384x over previous
import functools

import jax
import jax.numpy as jnp
from jax import lax
from jax.experimental import pallas as pl
from jax.experimental.pallas import tpu as pltpu
from jax.experimental.pallas import tpu_sc as plsc

B, S, H, E = 4, 8192, 2048, 64
NC, NS = 2, 16
NW = NC * NS
WPB = NW // B
HC = H // WPB
NV = HC // 16
R = 128
S_SC = 4096
NCHUNK = S_SC // R
RT = 512
NJ = (S - S_SC) // RT


def _pool_body(hs, out, buf, accv, sem0, sem1):
    wid = lax.axis_index("s") * NC + lax.axis_index("c")
    b = wid // WPB
    h0 = (wid % WPB) * HC

    def copy_in(g, slot, sem):
        return pltpu.make_async_copy(
            hs.at[b, pl.ds(g * R, R), pl.ds(h0, HC)], buf.at[slot], sem)

    def acc_rows(slot, acc):
        def row_step(r, a):
            return tuple(a[h] + buf[slot, r, pl.ds(h * 16, 16)]
                         for h in range(NV))
        return lax.fori_loop(0, R, row_step, acc)

    copy_in(0, 0, sem0).start()
    acc0 = tuple(jnp.zeros((16,), jnp.float32) for _ in range(NV))

    def chunk_pair(i, acc):
        g = i * 2
        copy_in(g + 1, 1, sem1).start()
        copy_in(g, 0, sem0).wait()
        acc = acc_rows(0, acc)

        @pl.when(g + 2 < NCHUNK)
        def _():
            copy_in(g + 2, 0, sem0).start()

        copy_in(g + 1, 1, sem1).wait()
        acc = acc_rows(1, acc)
        return acc

    acc = lax.fori_loop(0, NCHUNK // 2, chunk_pair, acc0)
    for h in range(NV):
        accv[pl.ds(h * 16, 16)] = acc[h]
    pltpu.sync_copy(accv, out.at[b, pl.ds(h0, HC)])


_pool = functools.partial(
    pl.kernel,
    mesh=plsc.VectorSubcoreMesh(core_axis_name="c", subcore_axis_name="s"),
    out_type=jax.ShapeDtypeStruct((B, H), jnp.float32),
    scratch_types=[
        pltpu.VMEM((2, R, HC), jnp.float32),
        pltpu.VMEM((HC,), jnp.float32),
        pltpu.SemaphoreType.DMA,
        pltpu.SemaphoreType.DMA,
    ],
)(_pool_body)


def _tc_pool_body(x_ref, o_ref):
    @pl.when(pl.program_id(1) == 0)
    def _():
        o_ref[...] = jnp.zeros_like(o_ref)

    o_ref[...] += jnp.sum(x_ref[...], axis=1, keepdims=True)


_tc_pool = pl.pallas_call(
    _tc_pool_body,
    grid=(B, NJ),
    in_specs=[pl.BlockSpec((1, RT, H), lambda b, j: (b, S_SC // RT + j, 0))],
    out_specs=pl.BlockSpec((1, 1, H), lambda b, j: (b, 0, 0)),
    out_shape=jax.ShapeDtypeStruct((B, 1, H), jnp.float32),
)


def _tail_body(ps_sc_ref, ps_tc_ref, w_ref, bias_ref, temp_ref, noise_ref,
               gum_ref, sel_ref, probs_ref):
    pooled = (ps_sc_ref[...] + ps_tc_ref[:, 0, :]) * (1.0 / S)
    w = w_ref[...]
    logits = lax.dot_general(pooled, w, (((1,), (0,)), ((), ())),
                             preferred_element_type=jnp.float32)
    logits = logits + bias_ref[...] + noise_ref[...]
    t = temp_ref[0, 0]
    x = logits / t
    x = x - jnp.max(x, axis=-1, keepdims=True)
    ex = jnp.exp(x)
    probs = ex / jnp.sum(ex, axis=-1, keepdims=True)
    probs_ref[...] = probs
    y = jnp.log(probs + 1e-20) + gum_ref[...]
    idx = lax.broadcasted_iota(jnp.int32, y.shape, 1)
    m1 = jnp.max(y, axis=-1, keepdims=True)
    i1 = jnp.min(jnp.where(y == m1, idx, E), axis=-1, keepdims=True)
    y2 = jnp.where(idx == i1, -jnp.inf, y)
    m2 = jnp.max(y2, axis=-1, keepdims=True)
    i2 = jnp.min(jnp.where(y2 == m2, idx, E), axis=-1, keepdims=True)
    sel_ref[...] = jnp.concatenate([i1, i2], axis=-1)


_tail = pl.pallas_call(
    _tail_body,
    out_shape=(jax.ShapeDtypeStruct((B, 2), jnp.int32),
               jax.ShapeDtypeStruct((B, E), jnp.float32)),
    in_specs=[
        pl.BlockSpec(memory_space=pltpu.VMEM),
        pl.BlockSpec(memory_space=pltpu.VMEM),
        pl.BlockSpec(memory_space=pltpu.VMEM),
        pl.BlockSpec(memory_space=pltpu.VMEM),
        pl.BlockSpec(memory_space=pltpu.SMEM),
        pl.BlockSpec(memory_space=pltpu.VMEM),
        pl.BlockSpec(memory_space=pltpu.VMEM),
    ],
)


def kernel(hidden_states, W, b, temperature, diversity_bonus):
    noise_key = jax.random.fold_in(jax.random.key(0), 1)
    noise = jax.random.normal(noise_key, (B, E), jnp.float32) * diversity_bonus
    g_key = jax.random.fold_in(jax.random.key(0), 2)
    gumbel = jax.random.gumbel(g_key, (B, E), jnp.float32)
    ps_sc = _pool(hidden_states)
    ps_tc = _tc_pool(hidden_states)
    sel, probs = _tail(ps_sc, ps_tc, W, b.reshape(1, E),
                       temperature.reshape(1, 1), noise, gumbel)
    return sel, probs


# speedup vs baseline: 1.2403x; 1.0015x over previous
import functools

import jax
import jax.numpy as jnp
from jax import lax
from jax.experimental import pallas as pl
from jax.experimental.pallas import tpu as pltpu
from jax.experimental.pallas import tpu_sc as plsc

B, S, H, E = 4, 8192, 2048, 64
NC, NS = 2, 16
NW = NC * NS
WPB = NW // B
HC = H // WPB
NV = HC // 16
R = 128
S_SC = 3072
NCHUNK = S_SC // R
RT = 512
NJ = (S - S_SC) // RT


def _pool_body(hs, out, buf, accv, sem0, sem1):
    wid = lax.axis_index("s") * NC + lax.axis_index("c")
    b = wid // WPB
    h0 = (wid % WPB) * HC

    def copy_in(g, slot, sem):
        return pltpu.make_async_copy(
            hs.at[b, pl.ds(g * R, R), pl.ds(h0, HC)], buf.at[slot], sem)

    def acc_rows(slot, acc):
        def row_step(r, a):
            return tuple(a[h] + buf[slot, r, pl.ds(h * 16, 16)]
                         for h in range(NV))
        return lax.fori_loop(0, R, row_step, acc)

    copy_in(0, 0, sem0).start()
    acc0 = tuple(jnp.zeros((16,), jnp.float32) for _ in range(NV))

    def chunk_pair(i, acc):
        g = i * 2
        copy_in(g + 1, 1, sem1).start()
        copy_in(g, 0, sem0).wait()
        acc = acc_rows(0, acc)

        @pl.when(g + 2 < NCHUNK)
        def _():
            copy_in(g + 2, 0, sem0).start()

        copy_in(g + 1, 1, sem1).wait()
        acc = acc_rows(1, acc)
        return acc

    acc = lax.fori_loop(0, NCHUNK // 2, chunk_pair, acc0)
    for h in range(NV):
        accv[pl.ds(h * 16, 16)] = acc[h]
    pltpu.sync_copy(accv, out.at[b, pl.ds(h0, HC)])


_pool = functools.partial(
    pl.kernel,
    mesh=plsc.VectorSubcoreMesh(core_axis_name="c", subcore_axis_name="s"),
    out_type=jax.ShapeDtypeStruct((B, H), jnp.float32),
    scratch_types=[
        pltpu.VMEM((2, R, HC), jnp.float32),
        pltpu.VMEM((HC,), jnp.float32),
        pltpu.SemaphoreType.DMA,
        pltpu.SemaphoreType.DMA,
    ],
)(_pool_body)


def _tc_pool_body(x_ref, o_ref):
    @pl.when(pl.program_id(0) == 0)
    def _():
        o_ref[...] = jnp.zeros_like(o_ref)

    o_ref[...] += jnp.sum(x_ref[...], axis=1, keepdims=True)


_tc_pool = pl.pallas_call(
    _tc_pool_body,
    grid=(NJ,),
    in_specs=[pl.BlockSpec((B, RT, H), lambda j: (0, S_SC // RT + j, 0))],
    out_specs=pl.BlockSpec((B, 1, H), lambda j: (0, 0, 0)),
    out_shape=jax.ShapeDtypeStruct((B, 1, H), jnp.float32),
)


def _tail_body(ps_sc_ref, ps_tc_ref, w_ref, bias_ref, temp_ref, noise_ref,
               gum_ref, sel_ref, probs_ref):
    pooled = (ps_sc_ref[...] + ps_tc_ref[:, 0, :]) * (1.0 / S)
    w = w_ref[...]
    logits = lax.dot_general(pooled, w, (((1,), (0,)), ((), ())),
                             preferred_element_type=jnp.float32)
    logits = logits + bias_ref[...] + noise_ref[...]
    t = temp_ref[0, 0]
    x = logits / t
    x = x - jnp.max(x, axis=-1, keepdims=True)
    ex = jnp.exp(x)
    probs = ex / jnp.sum(ex, axis=-1, keepdims=True)
    probs_ref[...] = probs
    y = jnp.log(probs + 1e-20) + gum_ref[...]
    idx = lax.broadcasted_iota(jnp.int32, y.shape, 1)
    m1 = jnp.max(y, axis=-1, keepdims=True)
    i1 = jnp.min(jnp.where(y == m1, idx, E), axis=-1, keepdims=True)
    y2 = jnp.where(idx == i1, -jnp.inf, y)
    m2 = jnp.max(y2, axis=-1, keepdims=True)
    i2 = jnp.min(jnp.where(y2 == m2, idx, E), axis=-1, keepdims=True)
    sel_ref[...] = jnp.concatenate([i1, i2], axis=-1)


_tail = pl.pallas_call(
    _tail_body,
    out_shape=(jax.ShapeDtypeStruct((B, 2), jnp.int32),
               jax.ShapeDtypeStruct((B, E), jnp.float32)),
    in_specs=[
        pl.BlockSpec(memory_space=pltpu.VMEM),
        pl.BlockSpec(memory_space=pltpu.VMEM),
        pl.BlockSpec(memory_space=pltpu.VMEM),
        pl.BlockSpec(memory_space=pltpu.VMEM),
        pl.BlockSpec(memory_space=pltpu.SMEM),
        pl.BlockSpec(memory_space=pltpu.VMEM),
        pl.BlockSpec(memory_space=pltpu.VMEM),
    ],
)


def kernel(hidden_states, W, b, temperature, diversity_bonus):
    noise_key = jax.random.fold_in(jax.random.key(0), 1)
    noise = jax.random.normal(noise_key, (B, E), jnp.float32) * diversity_bonus
    g_key = jax.random.fold_in(jax.random.key(0), 2)
    gumbel = jax.random.gumbel(g_key, (B, E), jnp.float32)
    ps_sc = _pool(hidden_states)
    ps_tc = _tc_pool(hidden_states)
    sel, probs = _tail(ps_sc, ps_tc, W, b.reshape(1, E),
                       temperature.reshape(1, 1), noise, gumbel)
    return sel, probs


# speedup vs baseline: 1.2655x; 1.0204x over previous
import functools

import jax
import jax.numpy as jnp
from jax import lax
from jax.experimental import pallas as pl
from jax.experimental.pallas import tpu as pltpu
from jax.experimental.pallas import tpu_sc as plsc

B, S, H, E = 4, 8192, 2048, 64
NC, NS = 2, 16
NW = NC * NS
WPB = NW // B
HC = H // WPB
NV = HC // 16
R = 128
S_SC = 1024
NCHUNK = S_SC // R
RT = 512
NJ = (S - S_SC) // RT


def _pool_body(hs, out, buf, accv, sem0, sem1):
    wid = lax.axis_index("s") * NC + lax.axis_index("c")
    b = wid // WPB
    h0 = (wid % WPB) * HC

    def copy_in(g, slot, sem):
        return pltpu.make_async_copy(
            hs.at[b, pl.ds(g * R, R), pl.ds(h0, HC)], buf.at[slot], sem)

    def acc_rows(slot, acc):
        def row_step(r, a):
            return tuple(a[h] + buf[slot, r, pl.ds(h * 16, 16)]
                         for h in range(NV))
        return lax.fori_loop(0, R, row_step, acc)

    copy_in(0, 0, sem0).start()
    acc0 = tuple(jnp.zeros((16,), jnp.float32) for _ in range(NV))

    def chunk_pair(i, acc):
        g = i * 2
        copy_in(g + 1, 1, sem1).start()
        copy_in(g, 0, sem0).wait()
        acc = acc_rows(0, acc)

        @pl.when(g + 2 < NCHUNK)
        def _():
            copy_in(g + 2, 0, sem0).start()

        copy_in(g + 1, 1, sem1).wait()
        acc = acc_rows(1, acc)
        return acc

    acc = lax.fori_loop(0, NCHUNK // 2, chunk_pair, acc0)
    for h in range(NV):
        accv[pl.ds(h * 16, 16)] = acc[h]
    pltpu.sync_copy(accv, out.at[b, pl.ds(h0, HC)])


_pool = functools.partial(
    pl.kernel,
    mesh=plsc.VectorSubcoreMesh(core_axis_name="c", subcore_axis_name="s"),
    out_type=jax.ShapeDtypeStruct((B, H), jnp.float32),
    scratch_types=[
        pltpu.VMEM((2, R, HC), jnp.float32),
        pltpu.VMEM((HC,), jnp.float32),
        pltpu.SemaphoreType.DMA,
        pltpu.SemaphoreType.DMA,
    ],
)(_pool_body)


def _tc_pool_body(x_ref, o_ref):
    @pl.when(pl.program_id(0) == 0)
    def _():
        o_ref[...] = jnp.zeros_like(o_ref)

    o_ref[...] += jnp.sum(x_ref[...], axis=1, keepdims=True)


_tc_pool = pl.pallas_call(
    _tc_pool_body,
    grid=(NJ,),
    in_specs=[pl.BlockSpec((B, RT, H), lambda j: (0, S_SC // RT + j, 0))],
    out_specs=pl.BlockSpec((B, 1, H), lambda j: (0, 0, 0)),
    out_shape=jax.ShapeDtypeStruct((B, 1, H), jnp.float32),
)


def _tail_body(ps_sc_ref, ps_tc_ref, w_ref, bias_ref, temp_ref, noise_ref,
               gum_ref, sel_ref, probs_ref):
    pooled = (ps_sc_ref[...] + ps_tc_ref[:, 0, :]) * (1.0 / S)
    w = w_ref[...]
    logits = lax.dot_general(pooled, w, (((1,), (0,)), ((), ())),
                             preferred_element_type=jnp.float32)
    logits = logits + bias_ref[...] + noise_ref[...]
    t = temp_ref[0, 0]
    x = logits / t
    x = x - jnp.max(x, axis=-1, keepdims=True)
    ex = jnp.exp(x)
    probs = ex / jnp.sum(ex, axis=-1, keepdims=True)
    probs_ref[...] = probs
    y = jnp.log(probs + 1e-20) + gum_ref[...]
    idx = lax.broadcasted_iota(jnp.int32, y.shape, 1)
    m1 = jnp.max(y, axis=-1, keepdims=True)
    i1 = jnp.min(jnp.where(y == m1, idx, E), axis=-1, keepdims=True)
    y2 = jnp.where(idx == i1, -jnp.inf, y)
    m2 = jnp.max(y2, axis=-1, keepdims=True)
    i2 = jnp.min(jnp.where(y2 == m2, idx, E), axis=-1, keepdims=True)
    sel_ref[...] = jnp.concatenate([i1, i2], axis=-1)


_tail = pl.pallas_call(
    _tail_body,
    out_shape=(jax.ShapeDtypeStruct((B, 2), jnp.int32),
               jax.ShapeDtypeStruct((B, E), jnp.float32)),
    in_specs=[
        pl.BlockSpec(memory_space=pltpu.VMEM),
        pl.BlockSpec(memory_space=pltpu.VMEM),
        pl.BlockSpec(memory_space=pltpu.VMEM),
        pl.BlockSpec(memory_space=pltpu.VMEM),
        pl.BlockSpec(memory_space=pltpu.SMEM),
        pl.BlockSpec(memory_space=pltpu.VMEM),
        pl.BlockSpec(memory_space=pltpu.VMEM),
    ],
)


def kernel(hidden_states, W, b, temperature, diversity_bonus):
    noise_key = jax.random.fold_in(jax.random.key(0), 1)
    noise = jax.random.normal(noise_key, (B, E), jnp.float32) * diversity_bonus
    g_key = jax.random.fold_in(jax.random.key(0), 2)
    gumbel = jax.random.gumbel(g_key, (B, E), jnp.float32)
    ps_sc = _pool(hidden_states)
    ps_tc = _tc_pool(hidden_states)
    sel, probs = _tail(ps_sc, ps_tc, W, b.reshape(1, E),
                       temperature.reshape(1, 1), noise, gumbel)
    return sel, probs


# speedup vs baseline: 1.4989x; 1.1844x over previous
import functools

import jax
import jax.numpy as jnp
from jax import lax
from jax.experimental import pallas as pl
from jax.experimental.pallas import tpu as pltpu
from jax.experimental.pallas import tpu_sc as plsc

B, S, H, E = 4, 8192, 2048, 64
NC, NS = 2, 16
NW = NC * NS
WPB = NW // B
HC = H // WPB
NV = HC // 16
R = 128
S_SC = 0
NCHUNK = S_SC // R
RT = 512
NJ = (S - S_SC) // RT


def _pool_body(hs, out, buf, accv, sem0, sem1):
    wid = lax.axis_index("s") * NC + lax.axis_index("c")
    b = wid // WPB
    h0 = (wid % WPB) * HC

    def copy_in(g, slot, sem):
        return pltpu.make_async_copy(
            hs.at[b, pl.ds(g * R, R), pl.ds(h0, HC)], buf.at[slot], sem)

    def acc_rows(slot, acc):
        def row_step(r, a):
            return tuple(a[h] + buf[slot, r, pl.ds(h * 16, 16)]
                         for h in range(NV))
        return lax.fori_loop(0, R, row_step, acc)

    copy_in(0, 0, sem0).start()
    acc0 = tuple(jnp.zeros((16,), jnp.float32) for _ in range(NV))

    def chunk_pair(i, acc):
        g = i * 2
        copy_in(g + 1, 1, sem1).start()
        copy_in(g, 0, sem0).wait()
        acc = acc_rows(0, acc)

        @pl.when(g + 2 < NCHUNK)
        def _():
            copy_in(g + 2, 0, sem0).start()

        copy_in(g + 1, 1, sem1).wait()
        acc = acc_rows(1, acc)
        return acc

    acc = lax.fori_loop(0, NCHUNK // 2, chunk_pair, acc0)
    for h in range(NV):
        accv[pl.ds(h * 16, 16)] = acc[h]
    pltpu.sync_copy(accv, out.at[b, pl.ds(h0, HC)])


_pool = functools.partial(
    pl.kernel,
    mesh=plsc.VectorSubcoreMesh(core_axis_name="c", subcore_axis_name="s"),
    out_type=jax.ShapeDtypeStruct((B, H), jnp.float32),
    scratch_types=[
        pltpu.VMEM((2, R, HC), jnp.float32),
        pltpu.VMEM((HC,), jnp.float32),
        pltpu.SemaphoreType.DMA,
        pltpu.SemaphoreType.DMA,
    ],
)(_pool_body)


def _tc_pool_body(x_ref, o_ref):
    @pl.when(pl.program_id(0) == 0)
    def _():
        o_ref[...] = jnp.zeros_like(o_ref)

    o_ref[...] += jnp.sum(x_ref[...], axis=1, keepdims=True)


_tc_pool = pl.pallas_call(
    _tc_pool_body,
    grid=(NJ,),
    in_specs=[pl.BlockSpec((B, RT, H), lambda j: (0, S_SC // RT + j, 0))],
    out_specs=pl.BlockSpec((B, 1, H), lambda j: (0, 0, 0)),
    out_shape=jax.ShapeDtypeStruct((B, 1, H), jnp.float32),
)


def _tail_body(ps_sc_ref, ps_tc_ref, w_ref, bias_ref, temp_ref, noise_ref,
               gum_ref, sel_ref, probs_ref):
    pooled = (ps_sc_ref[...] + ps_tc_ref[:, 0, :]) * (1.0 / S)
    w = w_ref[...]
    logits = lax.dot_general(pooled, w, (((1,), (0,)), ((), ())),
                             preferred_element_type=jnp.float32)
    logits = logits + bias_ref[...] + noise_ref[...]
    t = temp_ref[0, 0]
    x = logits / t
    x = x - jnp.max(x, axis=-1, keepdims=True)
    ex = jnp.exp(x)
    probs = ex / jnp.sum(ex, axis=-1, keepdims=True)
    probs_ref[...] = probs
    y = jnp.log(probs + 1e-20) + gum_ref[...]
    idx = lax.broadcasted_iota(jnp.int32, y.shape, 1)
    m1 = jnp.max(y, axis=-1, keepdims=True)
    i1 = jnp.min(jnp.where(y == m1, idx, E), axis=-1, keepdims=True)
    y2 = jnp.where(idx == i1, -jnp.inf, y)
    m2 = jnp.max(y2, axis=-1, keepdims=True)
    i2 = jnp.min(jnp.where(y2 == m2, idx, E), axis=-1, keepdims=True)
    sel_ref[...] = jnp.concatenate([i1, i2], axis=-1)


_tail = pl.pallas_call(
    _tail_body,
    out_shape=(jax.ShapeDtypeStruct((B, 2), jnp.int32),
               jax.ShapeDtypeStruct((B, E), jnp.float32)),
    in_specs=[
        pl.BlockSpec(memory_space=pltpu.VMEM),
        pl.BlockSpec(memory_space=pltpu.VMEM),
        pl.BlockSpec(memory_space=pltpu.VMEM),
        pl.BlockSpec(memory_space=pltpu.VMEM),
        pl.BlockSpec(memory_space=pltpu.SMEM),
        pl.BlockSpec(memory_space=pltpu.VMEM),
        pl.BlockSpec(memory_space=pltpu.VMEM),
    ],
)


def kernel(hidden_states, W, b, temperature, diversity_bonus):
    noise_key = jax.random.fold_in(jax.random.key(0), 1)
    noise = jax.random.normal(noise_key, (B, E), jnp.float32) * diversity_bonus
    g_key = jax.random.fold_in(jax.random.key(0), 2)
    gumbel = jax.random.gumbel(g_key, (B, E), jnp.float32)
    ps_sc = jnp.zeros((B, H), jnp.float32)
    ps_tc = _tc_pool(hidden_states)
    sel, probs = _tail(ps_sc, ps_tc, W, b.reshape(1, E),
                       temperature.reshape(1, 1), noise, gumbel)
    return sel, probs
